# baseline (speedup 1.0000x reference)
import jax
import jax.numpy as jnp
from jax import lax
from jax.experimental import pallas as pl
from jax.experimental.pallas import tpu as pltpu
from jax.experimental.pallas import tpu_sc as plsc

B, S, D = 4, 4096, 1024
NC, NS = 2, 16
NW = NC * NS
SPW = S // NW
C = 8
G = SPW // C
NH = 2
DH = D // NH
NI = G * NH
NB = 6
NP = 4
LOOK = 3


def _pos_add_body(emb_hbm, pos_hbm, out_hbm, *refs):
    ebufs = [[refs[k * B + b] for b in range(B)] for k in range(NB)]
    pbufs = list(refs[NB * B:NB * B + NP])
    sems = refs[NB * B + NP:]
    se = sems[:NB]
    so = sems[NB:2 * NB]
    sp = sems[2 * NB:2 * NB + NP]

    wid = lax.axis_index("s") * NC + lax.axis_index("c")
    seq_base = wid * SPW

    de, dp, do = {}, {}, {}

    def issue_in(i):
        g, h = divmod(i, NH)
        k = i % NB
        row = seq_base + g * C
        col = h * DH
        de[i] = [
            pltpu.async_copy(
                emb_hbm.at[b, pl.ds(row, C), pl.ds(col, DH)],
                ebufs[k][b], se[k])
            for b in range(B)
        ]
        dp[i] = pltpu.async_copy(
            pos_hbm.at[pl.ds(row, C), pl.ds(col, DH)],
            pbufs[i % NP], sp[i % NP])

    for i in range(LOOK):
        issue_in(i)

    for i in range(NI):
        g, h = divmod(i, NH)
        k = i % NB
        for d in de[i]:
            d.wait()
        dp[i].wait()
        if i >= LOOK:
            for d in do[i - LOOK]:
                d.wait()
        if i + LOOK < NI:
            issue_in(i + LOOK)
        eb, pb = ebufs[k], pbufs[i % NP]

        def row_add(r, carry):
            @plsc.parallel_loop(0, DH, 16, unroll=4)
            def _(c):
                sl = pl.ds(c, 16)
                pv = pb[r, sl]
                for b in range(B):
                    eb[b][r, sl] = eb[b][r, sl] + pv
            return carry

        lax.fori_loop(0, C, row_add, 0)

        row = seq_base + g * C
        col = h * DH
        do[i] = [
            pltpu.async_copy(
                ebufs[k][b],
                out_hbm.at[b, pl.ds(row, C), pl.ds(col, DH)], so[k])
            for b in range(B)
        ]

    for i in range(max(0, NI - LOOK), NI):
        for d in do[i]:
            d.wait()


@jax.jit
def _run(embeddings, pos_table):
    f = pl.kernel(
        _pos_add_body,
        out_type=jax.ShapeDtypeStruct((B, S, D), jnp.float32),
        mesh=plsc.VectorSubcoreMesh(
            core_axis_name="c", subcore_axis_name="s",
            num_cores=NC, num_subcores=NS,
        ),
        scratch_types=(
            [pltpu.VMEM((C, DH), jnp.float32)] * (NB * B + NP)
            + [pltpu.SemaphoreType.DMA] * (2 * NB + NP)
        ),
        compiler_params=pltpu.CompilerParams(use_tc_tiling_on_sc=True),
    )
    return f(embeddings, pos_table)


def kernel(embeddings, pos_table):
    return _run(embeddings, pos_table)


# speedup vs baseline: 1.0295x; 1.0295x over previous
import jax
import jax.numpy as jnp
from jax import lax
from jax.experimental import pallas as pl
from jax.experimental.pallas import tpu as pltpu
from jax.experimental.pallas import tpu_sc as plsc

B, S, D = 4, 4096, 1024
NC, NS = 2, 16
NW = NC * NS
SPW = S // NW
C = 8
G = SPW // C
NPAIR = 2
NPG = B // NPAIR
NI = G * NPG
NB = 6
NP = 3
LOOK = 2


def _pos_add_body(emb_hbm, pos_hbm, out_hbm, *refs):
    ebufs = [[refs[k * NPAIR + b] for b in range(NPAIR)] for k in range(NB)]
    pbufs = list(refs[NB * NPAIR:NB * NPAIR + NP])
    sems = refs[NB * NPAIR + NP:]
    se = sems[:NB]
    so = sems[NB:2 * NB]
    sp = sems[2 * NB:2 * NB + NP]

    wid = lax.axis_index("s") * NC + lax.axis_index("c")
    seq_base = wid * SPW

    de, dp, do = {}, {}, {}

    def issue_in(i):
        g, p = divmod(i, NPG)
        k = i % NB
        row = seq_base + g * C
        de[i] = [
            pltpu.async_copy(
                emb_hbm.at[p * NPAIR + b, pl.ds(row, C), :],
                ebufs[k][b], se[k])
            for b in range(NPAIR)
        ]
        if p == 0:
            dp[g] = pltpu.async_copy(
                pos_hbm.at[pl.ds(row, C), :], pbufs[g % NP], sp[g % NP])

    for i in range(LOOK):
        issue_in(i)

    for i in range(NI):
        g, p = divmod(i, NPG)
        k = i % NB
        if i >= LOOK:
            for d in do[i - LOOK]:
                d.wait()
        if i + LOOK < NI:
            issue_in(i + LOOK)
        for d in de[i]:
            d.wait()
        if p == 0:
            dp[g].wait()
        eb, pb = ebufs[k], pbufs[g % NP]

        def row_add(r, carry):
            @plsc.parallel_loop(0, D, 16, unroll=4)
            def _(c):
                sl = pl.ds(c, 16)
                pv = pb[r, sl]
                for b in range(NPAIR):
                    eb[b][r, sl] = eb[b][r, sl] + pv
            return carry

        lax.fori_loop(0, C, row_add, 0)

        row = seq_base + g * C
        do[i] = [
            pltpu.async_copy(
                ebufs[k][b],
                out_hbm.at[p * NPAIR + b, pl.ds(row, C), :], so[k])
            for b in range(NPAIR)
        ]

    for i in range(max(0, NI - LOOK), NI):
        for d in do[i]:
            d.wait()


@jax.jit
def _run(embeddings, pos_table):
    f = pl.kernel(
        _pos_add_body,
        out_type=jax.ShapeDtypeStruct((B, S, D), jnp.float32),
        mesh=plsc.VectorSubcoreMesh(
            core_axis_name="c", subcore_axis_name="s",
            num_cores=NC, num_subcores=NS,
        ),
        scratch_types=(
            [pltpu.VMEM((C, D), jnp.float32)] * (NB * NPAIR + NP)
            + [pltpu.SemaphoreType.DMA] * (2 * NB + NP)
        ),
        compiler_params=pltpu.CompilerParams(use_tc_tiling_on_sc=True),
    )
    return f(embeddings, pos_table)


def kernel(embeddings, pos_table):
    return _run(embeddings, pos_table)
